# baseline (speedup 1.0000x reference)
import functools

import jax
import jax.numpy as jnp
from jax import lax
from jax.experimental import pallas as pl
from jax.experimental.pallas import tpu as pltpu
from jax.experimental.pallas import tpu_sc as plsc

E = 8
K = 2
N = 4096
D = 1024
F = 2048

T = 256
M_PAD = N * K + E * T
NT = M_PAD // T

NC, NS, L = 2, 16, 16
NW = NC * NS

_MESH = plsc.VectorSubcoreMesh(core_axis_name="c", subcore_axis_name="s")

T_PER_W = N // NW
DCH = 32
DN = T_PER_W // DCH
CT = 16
CN = T_PER_W // CT


@functools.partial(
    pl.kernel,
    mesh=_MESH,
    out_type=(jax.ShapeDtypeStruct((M_PAD, D), jnp.float32),
              jax.ShapeDtypeStruct((M_PAD, 128), jnp.float32)),
    scratch_types=[
        pltpu.VMEM((DCH, D), jnp.float32),
        pltpu.VMEM((DCH, D), jnp.float32),
        pltpu.VMEM((DCH, 128), jnp.float32),
        pltpu.VMEM((DCH, 128), jnp.float32),
        pltpu.VMEM((DCH, 16), jnp.float32),
        pltpu.VMEM((DCH, 16), jnp.float32),
        pltpu.VMEM((DCH,), jnp.int32),
        pltpu.VMEM((DCH,), jnp.int32),
        pltpu.VMEM((DCH,), jnp.int32),
        pltpu.VMEM((DCH,), jnp.int32),
        pltpu.SemaphoreType.DMA,
        pltpu.SemaphoreType.DMA,
        pltpu.SemaphoreType.DMA,
        pltpu.SemaphoreType.DMA,
    ],
)
def _sc_dispatch(x_hbm, w0b_hbm, w1b_hbm, pos0_hbm, pos1_hbm,
                 xs_hbm, wpad_hbm,
                 xb0, xb1, wb0, wb1, wsp0, wsp1, i00, i01, i10, i11,
                 gs0, gs1, ss0, ss1):
    wid = lax.axis_index("s") * NC + lax.axis_index("c")
    base = wid * T_PER_W
    xbufs = (xb0, xb1)
    wbufs = (wb0, wb1)
    wsps = (wsp0, wsp1)
    i0s = (i00, i01)
    i1s = (i10, i11)
    gsem = (gs0, gs1)
    ssem = (ss0, ss1)

    def fire(c):
        p = c % 2
        off = base + c * DCH
        cpx = pltpu.async_copy(x_hbm.at[pl.ds(off, DCH)], xbufs[p], gsem[p])
        pltpu.sync_copy(pos0_hbm.at[pl.ds(off, DCH)], i0s[p])
        pltpu.sync_copy(pos1_hbm.at[pl.ds(off, DCH)], i1s[p])
        pltpu.sync_copy(w0b_hbm.at[pl.ds(off, DCH)], wsps[p])
        return cpx

    gcp = [None] * DN
    scp = [None] * DN
    gcp[0] = fire(0)
    for c in range(DN):
        p = c % 2
        off = base + c * DCH

        def _wrow0(i, _):
            v = wsps[p][i, :]
            def _st(j, _):
                wbufs[p][i, pl.ds(j * L, L)] = v
                return 0
            return lax.fori_loop(0, 128 // L, _st, 0, unroll=8)
        lax.fori_loop(0, DCH, _wrow0, 0)

        if c + 1 < DN:
            if c >= 1:
                scp[c - 1][3].wait()
                scp[c - 1][1].wait()
            gcp[c + 1] = fire(c + 1)
        gcp[c].wait()
        s_x0 = pltpu.async_copy(xbufs[p], xs_hbm.at[i0s[p]], ssem[p])
        s_w0 = pltpu.async_copy(wbufs[p], wpad_hbm.at[i0s[p]], gsem[p])
        s_x0.wait()
        s_x1 = pltpu.async_copy(xbufs[p], xs_hbm.at[i1s[p]], ssem[p])
        pltpu.sync_copy(w1b_hbm.at[pl.ds(off, DCH)], wsps[p])
        s_w0.wait()

        def _wrow1(i, _):
            v = wsps[p][i, :]
            def _st(j, _):
                wbufs[p][i, pl.ds(j * L, L)] = v
                return 0
            return lax.fori_loop(0, 128 // L, _st, 0, unroll=8)
        lax.fori_loop(0, DCH, _wrow1, 0)

        s_w1 = pltpu.async_copy(wbufs[p], wpad_hbm.at[i1s[p]], gsem[p])
        scp[c] = (None, s_x1, None, s_w1)
    scp[DN - 1][1].wait()
    scp[DN - 1][3].wait()
    if DN >= 2:
        scp[DN - 2][1].wait()
        scp[DN - 2][3].wait()


@functools.partial(
    pl.kernel,
    mesh=_MESH,
    out_type=jax.ShapeDtypeStruct((N, D), jnp.float32),
    scratch_types=[
        pltpu.VMEM((T_PER_W,), jnp.int32),
        pltpu.VMEM((T_PER_W,), jnp.int32),
        pltpu.VMEM((CT, D), jnp.float32),
        pltpu.VMEM((CT, D), jnp.float32),
        pltpu.VMEM((CT, D), jnp.float32),
        pltpu.VMEM((CT, D), jnp.float32),
        pltpu.SemaphoreType.DMA,
        pltpu.SemaphoreType.DMA,
        pltpu.SemaphoreType.DMA,
        pltpu.SemaphoreType.DMA,
        pltpu.SemaphoreType.DMA,
        pltpu.SemaphoreType.DMA,
    ],
)
def _sc_combine(ys_hbm, pos0_hbm, pos1_hbm, out_hbm, i0_v, i1_v,
                a0, a1, b0, b1, ga0, ga1, gb0, gb1, st0, st1):
    wid = lax.axis_index("s") * NC + lax.axis_index("c")
    base = wid * T_PER_W
    pltpu.sync_copy(pos0_hbm.at[pl.ds(base, T_PER_W)], i0_v)
    pltpu.sync_copy(pos1_hbm.at[pl.ds(base, T_PER_W)], i1_v)
    abufs = (a0, a1)
    bbufs = (b0, b1)
    gasem = (ga0, ga1)
    gbsem = (gb0, gb1)
    ssem = (st0, st1)

    def fire(c):
        p = c % 2
        cpa = pltpu.async_copy(
            ys_hbm.at[i0_v.at[pl.ds(c * CT, CT)]], abufs[p], gasem[p])
        cpb = pltpu.async_copy(
            ys_hbm.at[i1_v.at[pl.ds(c * CT, CT)]], bbufs[p], gbsem[p])
        return cpa, cpb

    gcp = [None] * CN
    scp = [None] * CN
    gcp[0] = fire(0)
    for c in range(CN):
        p = c % 2
        if c + 1 < CN:
            if c >= 1:
                scp[c - 1].wait()
            gcp[c + 1] = fire(c + 1)
        gcp[c][0].wait()
        gcp[c][1].wait()

        def _row(i, _):
            def _lane(j, _):
                abufs[p][i, pl.ds(j * L, L)] = (
                    abufs[p][i, pl.ds(j * L, L)] + bbufs[p][i, pl.ds(j * L, L)])
                return 0
            return lax.fori_loop(0, D // L, _lane, 0, unroll=4)

        lax.fori_loop(0, CT, _row, 0)
        scp[c] = pltpu.async_copy(
            abufs[p], out_hbm.at[pl.ds(base + c * CT, CT)], ssem[p])
    scp[CN - 1].wait()
    if CN >= 2:
        scp[CN - 2].wait()


NBUF = 2
NSPLIT = 4


def _ffn_body(eid_ref, x_ref, w_ref, w1_any, w2_any, o_ref,
              w1buf, w2buf, w1b16, w2b16, sem1, sem2, st_ref):
    i = pl.program_id(0)
    e = eid_ref[i]
    prev_e = eid_ref[jnp.maximum(i - 1, 0)]
    first = jnp.logical_or(i == 0, e != prev_e)

    @pl.when(i == 0)
    def _():
        st_ref[0] = 0

    horizon = jnp.minimum(e + (NBUF - 1), E - 1)
    for f in range(E):
        @pl.when(jnp.logical_and(f >= st_ref[0], f <= horizon))
        def _(f=f):
            for h in range(NSPLIT):
                pltpu.make_async_copy(
                    w1_any.at[f, pl.ds(h * (D // NSPLIT), D // NSPLIT)],
                    w1buf.at[f % NBUF, pl.ds(h * (D // NSPLIT), D // NSPLIT)],
                    sem1.at[f % NBUF]).start()
                pltpu.make_async_copy(
                    w2_any.at[f, pl.ds(h * (F // NSPLIT), F // NSPLIT)],
                    w2buf.at[f % NBUF, pl.ds(h * (F // NSPLIT), F // NSPLIT)],
                    sem2.at[f % NBUF]).start()
    st_ref[0] = jnp.maximum(st_ref[0], horizon + 1)

    @pl.when(first)
    def _():
        s = e % NBUF
        for h in range(NSPLIT):
            pltpu.make_async_copy(
                w1_any.at[e, pl.ds(h * (D // NSPLIT), D // NSPLIT)],
                w1buf.at[e % NBUF, pl.ds(h * (D // NSPLIT), D // NSPLIT)],
                sem1.at[e % NBUF]).wait()
        w1b16[...] = w1buf[s].astype(jnp.bfloat16)

    xb = (x_ref[...] * w_ref[:, 0:1]).astype(jnp.bfloat16)
    h = jnp.maximum(
        jnp.dot(xb, w1b16[...], preferred_element_type=jnp.float32), 0.0)

    @pl.when(first)
    def _():
        s = e % NBUF
        for h2 in range(NSPLIT):
            pltpu.make_async_copy(
                w2_any.at[e, pl.ds(h2 * (F // NSPLIT), F // NSPLIT)],
                w2buf.at[e % NBUF, pl.ds(h2 * (F // NSPLIT), F // NSPLIT)],
                sem2.at[e % NBUF]).wait()
        w2b16[...] = w2buf[s].astype(jnp.bfloat16)

    hb = h.astype(jnp.bfloat16)
    o_ref[...] = jnp.dot(hb, w2b16[...], preferred_element_type=jnp.float32)


_GRID_SPEC = pltpu.PrefetchScalarGridSpec(
    num_scalar_prefetch=1,
    grid=(NT,),
    in_specs=[
        pl.BlockSpec((T, D), lambda i, eid: (i, 0)),
        pl.BlockSpec((T, 128), lambda i, eid: (i, 0)),
        pl.BlockSpec(memory_space=pl.ANY),
        pl.BlockSpec(memory_space=pl.ANY),
    ],
    out_specs=pl.BlockSpec((T, D), lambda i, eid: (i, 0)),
    scratch_shapes=[
        pltpu.VMEM((NBUF, D, F), jnp.float32),
        pltpu.VMEM((NBUF, F, D), jnp.float32),
        pltpu.VMEM((D, F), jnp.bfloat16),
        pltpu.VMEM((F, D), jnp.bfloat16),
        pltpu.SemaphoreType.DMA((NBUF,)),
        pltpu.SemaphoreType.DMA((NBUF,)),
        pltpu.SMEM((1,), jnp.int32),
    ],
)

_ffn_call = pl.pallas_call(
    _ffn_body,
    grid_spec=_GRID_SPEC,
    out_shape=jax.ShapeDtypeStruct((M_PAD, D), jnp.float32),
)


def kernel(hidden_states, expert_indices, routing_weights, W1, W2):
    flat_e = expert_indices.reshape(-1).astype(jnp.int32)
    onehot_i = (flat_e[None, :] ==
                jnp.arange(E, dtype=jnp.int32)[:, None]).astype(jnp.int32)
    cum = jnp.cumsum(onehot_i, axis=1)
    counts = cum[:, -1]
    pcount = jnp.maximum((counts + T - 1) // T, 1) * T
    pends = jnp.cumsum(pcount).astype(jnp.int32)
    pstart = pends - pcount
    ppos = jnp.sum(onehot_i * (pstart[:, None] + cum - 1), axis=0)
    pos01 = ppos.reshape(N, K)
    pos0 = pos01[:, 0]
    pos1 = pos01[:, 1]
    tile_eid = jnp.minimum(
        jnp.sum((jnp.arange(NT, dtype=jnp.int32)[:, None] >=
                 (pends // T)[None, :]).astype(jnp.int32), axis=1),
        E - 1).astype(jnp.int32)
    w0b = jnp.broadcast_to(routing_weights[:, 0:1], (N, 16))
    w1b = jnp.broadcast_to(routing_weights[:, 1:2], (N, 16))

    xs, wpad = _sc_dispatch(hidden_states, w0b, w1b, pos0, pos1)
    ys = _ffn_call(tile_eid, xs, wpad, W1, W2)
    out = _sc_combine(ys, pos0, pos1)
    return out


# speedup vs baseline: 1.0394x; 1.0394x over previous
import functools

import jax
import jax.numpy as jnp
from jax import lax
from jax.experimental import pallas as pl
from jax.experimental.pallas import tpu as pltpu
from jax.experimental.pallas import tpu_sc as plsc

E = 8
K = 2
N = 4096
D = 1024
F = 2048

T = 256
M_PAD = N * K + E * T
NT = M_PAD // T

NC, NS, L = 2, 16, 16
NW = NC * NS

_MESH = plsc.VectorSubcoreMesh(core_axis_name="c", subcore_axis_name="s")

T_PER_W = N // NW
DCH = 32
DN = T_PER_W // DCH
CT = 16
CN = T_PER_W // CT


@functools.partial(
    pl.kernel,
    mesh=_MESH,
    out_type=(jax.ShapeDtypeStruct((M_PAD, D), jnp.float32),
              jax.ShapeDtypeStruct((M_PAD, 128), jnp.float32)),
    scratch_types=[
        pltpu.VMEM((DCH, D), jnp.float32),
        pltpu.VMEM((DCH, D), jnp.float32),
        pltpu.VMEM((DCH, 128), jnp.float32),
        pltpu.VMEM((DCH, 128), jnp.float32),
        pltpu.VMEM((DCH, 16), jnp.float32),
        pltpu.VMEM((DCH, 16), jnp.float32),
        pltpu.VMEM((DCH,), jnp.int32),
        pltpu.VMEM((DCH,), jnp.int32),
        pltpu.VMEM((DCH,), jnp.int32),
        pltpu.VMEM((DCH,), jnp.int32),
        pltpu.SemaphoreType.DMA,
        pltpu.SemaphoreType.DMA,
        pltpu.SemaphoreType.DMA,
        pltpu.SemaphoreType.DMA,
    ],
)
def _sc_dispatch(x_hbm, w0b_hbm, w1b_hbm, pos0_hbm, pos1_hbm,
                 xs_hbm, wpad_hbm,
                 xb0, xb1, wb0, wb1, wsp0, wsp1, i00, i01, i10, i11,
                 gs0, gs1, ss0, ss1):
    wid = lax.axis_index("s") * NC + lax.axis_index("c")
    base = wid * T_PER_W
    xbufs = (xb0, xb1)
    wbufs = (wb0, wb1)
    wsps = (wsp0, wsp1)
    i0s = (i00, i01)
    i1s = (i10, i11)
    gsem = (gs0, gs1)
    ssem = (ss0, ss1)

    def fire(c):
        p = c % 2
        off = base + c * DCH
        cpx = pltpu.async_copy(x_hbm.at[pl.ds(off, DCH)], xbufs[p], gsem[p])
        pltpu.sync_copy(pos0_hbm.at[pl.ds(off, DCH)], i0s[p])
        pltpu.sync_copy(pos1_hbm.at[pl.ds(off, DCH)], i1s[p])
        pltpu.sync_copy(w0b_hbm.at[pl.ds(off, DCH)], wsps[p])
        return cpx

    gcp = [None] * DN
    scp = [None] * DN
    gcp[0] = fire(0)
    for c in range(DN):
        p = c % 2
        off = base + c * DCH

        def _wrow0(i, _):
            v = wsps[p][i, :]
            def _st(j, _):
                wbufs[p][i, pl.ds(j * L, L)] = v
                return 0
            return lax.fori_loop(0, 128 // L, _st, 0, unroll=8)
        lax.fori_loop(0, DCH, _wrow0, 0)

        if c + 1 < DN:
            if c >= 1:
                scp[c - 1][3].wait()
                scp[c - 1][1].wait()
            gcp[c + 1] = fire(c + 1)
        gcp[c].wait()
        s_x0 = pltpu.async_copy(xbufs[p], xs_hbm.at[i0s[p]], ssem[p])
        s_w0 = pltpu.async_copy(wbufs[p], wpad_hbm.at[i0s[p]], gsem[p])
        s_x0.wait()
        s_x1 = pltpu.async_copy(xbufs[p], xs_hbm.at[i1s[p]], ssem[p])
        pltpu.sync_copy(w1b_hbm.at[pl.ds(off, DCH)], wsps[p])
        s_w0.wait()

        def _wrow1(i, _):
            v = wsps[p][i, :]
            def _st(j, _):
                wbufs[p][i, pl.ds(j * L, L)] = v
                return 0
            return lax.fori_loop(0, 128 // L, _st, 0, unroll=8)
        lax.fori_loop(0, DCH, _wrow1, 0)

        s_w1 = pltpu.async_copy(wbufs[p], wpad_hbm.at[i1s[p]], gsem[p])
        scp[c] = (None, s_x1, None, s_w1)
    scp[DN - 1][1].wait()
    scp[DN - 1][3].wait()
    if DN >= 2:
        scp[DN - 2][1].wait()
        scp[DN - 2][3].wait()


@functools.partial(
    pl.kernel,
    mesh=_MESH,
    out_type=jax.ShapeDtypeStruct((N, D), jnp.float32),
    scratch_types=[
        pltpu.VMEM((T_PER_W,), jnp.int32),
        pltpu.VMEM((T_PER_W,), jnp.int32),
        pltpu.VMEM((CT, D), jnp.float32),
        pltpu.VMEM((CT, D), jnp.float32),
        pltpu.VMEM((CT, D), jnp.float32),
        pltpu.VMEM((CT, D), jnp.float32),
        pltpu.SemaphoreType.DMA,
        pltpu.SemaphoreType.DMA,
        pltpu.SemaphoreType.DMA,
        pltpu.SemaphoreType.DMA,
        pltpu.SemaphoreType.DMA,
        pltpu.SemaphoreType.DMA,
    ],
)
def _sc_combine(ys_hbm, pos0_hbm, pos1_hbm, out_hbm, i0_v, i1_v,
                a0, a1, b0, b1, ga0, ga1, gb0, gb1, st0, st1):
    wid = lax.axis_index("s") * NC + lax.axis_index("c")
    base = wid * T_PER_W
    pltpu.sync_copy(pos0_hbm.at[pl.ds(base, T_PER_W)], i0_v)
    pltpu.sync_copy(pos1_hbm.at[pl.ds(base, T_PER_W)], i1_v)
    abufs = (a0, a1)
    bbufs = (b0, b1)
    gasem = (ga0, ga1)
    gbsem = (gb0, gb1)
    ssem = (st0, st1)

    def fire(c):
        p = c % 2
        cpa = pltpu.async_copy(
            ys_hbm.at[i0_v.at[pl.ds(c * CT, CT)]], abufs[p], gasem[p])
        cpb = pltpu.async_copy(
            ys_hbm.at[i1_v.at[pl.ds(c * CT, CT)]], bbufs[p], gbsem[p])
        return cpa, cpb

    gcp = [None] * CN
    scp = [None] * CN
    gcp[0] = fire(0)
    for c in range(CN):
        p = c % 2
        if c + 1 < CN:
            if c >= 1:
                scp[c - 1].wait()
            gcp[c + 1] = fire(c + 1)
        gcp[c][0].wait()
        gcp[c][1].wait()

        def _row(i, _):
            def _lane(j, _):
                abufs[p][i, pl.ds(j * L, L)] = (
                    abufs[p][i, pl.ds(j * L, L)] + bbufs[p][i, pl.ds(j * L, L)])
                return 0
            return lax.fori_loop(0, D // L, _lane, 0, unroll=4)

        lax.fori_loop(0, CT, _row, 0)
        scp[c] = pltpu.async_copy(
            abufs[p], out_hbm.at[pl.ds(base + c * CT, CT)], ssem[p])
    scp[CN - 1].wait()
    if CN >= 2:
        scp[CN - 2].wait()


NBUF = 2
NSPLIT = 4


def _ffn_body(eid_ref, x_ref, w_ref, w1_any, w2_any, o_ref,
              w1buf, w2buf, w1b16, w2b16, sem1, sem2, st_ref):
    i = pl.program_id(0)
    e = eid_ref[i]
    prev_e = eid_ref[jnp.maximum(i - 1, 0)]
    first = jnp.logical_or(i == 0, e != prev_e)

    @pl.when(i == 0)
    def _():
        st_ref[0] = 0

    horizon = jnp.minimum(e + (NBUF - 1), E - 1)
    for f in range(E):
        @pl.when(jnp.logical_and(f >= st_ref[0], f <= horizon))
        def _(f=f):
            for h in range(NSPLIT):
                pltpu.make_async_copy(
                    w1_any.at[f, pl.ds(h * (D // NSPLIT), D // NSPLIT)],
                    w1buf.at[f % NBUF, pl.ds(h * (D // NSPLIT), D // NSPLIT)],
                    sem1.at[f % NBUF]).start()
                pltpu.make_async_copy(
                    w2_any.at[f, pl.ds(h * (F // NSPLIT), F // NSPLIT)],
                    w2buf.at[f % NBUF, pl.ds(h * (F // NSPLIT), F // NSPLIT)],
                    sem2.at[f % NBUF]).start()
    st_ref[0] = jnp.maximum(st_ref[0], horizon + 1)

    @pl.when(first)
    def _():
        s = e % NBUF
        for h in range(NSPLIT):
            pltpu.make_async_copy(
                w1_any.at[e, pl.ds(h * (D // NSPLIT), D // NSPLIT)],
                w1buf.at[e % NBUF, pl.ds(h * (D // NSPLIT), D // NSPLIT)],
                sem1.at[e % NBUF]).wait()
            pltpu.make_async_copy(
                w2_any.at[e, pl.ds(h * (F // NSPLIT), F // NSPLIT)],
                w2buf.at[e % NBUF, pl.ds(h * (F // NSPLIT), F // NSPLIT)],
                sem2.at[e % NBUF]).wait()
        w1b16[...] = w1buf[s].astype(jnp.bfloat16)
        w2b16[...] = w2buf[s].astype(jnp.bfloat16)

    xb = (x_ref[...] * w_ref[:, 0:1]).astype(jnp.bfloat16)
    h = jnp.maximum(
        jnp.dot(xb, w1b16[...], preferred_element_type=jnp.float32), 0.0)
    hb = h.astype(jnp.bfloat16)
    o_ref[...] = jnp.dot(hb, w2b16[...], preferred_element_type=jnp.float32)


_GRID_SPEC = pltpu.PrefetchScalarGridSpec(
    num_scalar_prefetch=1,
    grid=(NT,),
    in_specs=[
        pl.BlockSpec((T, D), lambda i, eid: (i, 0)),
        pl.BlockSpec((T, 128), lambda i, eid: (i, 0)),
        pl.BlockSpec(memory_space=pl.ANY),
        pl.BlockSpec(memory_space=pl.ANY),
    ],
    out_specs=pl.BlockSpec((T, D), lambda i, eid: (i, 0)),
    scratch_shapes=[
        pltpu.VMEM((NBUF, D, F), jnp.float32),
        pltpu.VMEM((NBUF, F, D), jnp.float32),
        pltpu.VMEM((D, F), jnp.bfloat16),
        pltpu.VMEM((F, D), jnp.bfloat16),
        pltpu.SemaphoreType.DMA((NBUF,)),
        pltpu.SemaphoreType.DMA((NBUF,)),
        pltpu.SMEM((1,), jnp.int32),
    ],
)

_ffn_call = pl.pallas_call(
    _ffn_body,
    grid_spec=_GRID_SPEC,
    out_shape=jax.ShapeDtypeStruct((M_PAD, D), jnp.float32),
)


def kernel(hidden_states, expert_indices, routing_weights, W1, W2):
    flat_e = expert_indices.reshape(-1).astype(jnp.int32)
    onehot_f = (flat_e[None, :] ==
                jnp.arange(E, dtype=jnp.int32)[:, None]).astype(jnp.float32)
    oh3 = onehot_f.reshape(E, (N * K) // 128, 128)
    u128 = jnp.triu(jnp.ones((128, 128), jnp.float32))
    p_in = oh3 @ u128
    rowsum = p_in[:, :, -1]
    u64s = jnp.triu(jnp.ones(((N * K) // 128, (N * K) // 128),
                             jnp.float32), k=1)
    carry = rowsum @ u64s
    cum = (p_in + carry[:, :, None]).reshape(E, N * K)
    counts = cum[:, -1].astype(jnp.int32)
    pcount = jnp.maximum((counts + T - 1) // T, 1) * T
    pends = jnp.cumsum(pcount).astype(jnp.int32)
    pstart = pends - pcount
    ppos = jnp.sum(onehot_f * (pstart.astype(jnp.float32)[:, None] + cum - 1.0),
                   axis=0).astype(jnp.int32)
    pos01 = ppos.reshape(N, K)
    pos0 = pos01[:, 0]
    pos1 = pos01[:, 1]
    tile_eid = jnp.minimum(
        jnp.sum((jnp.arange(NT, dtype=jnp.int32)[:, None] >=
                 (pends // T)[None, :]).astype(jnp.int32), axis=1),
        E - 1).astype(jnp.int32)
    w0b = jnp.broadcast_to(routing_weights[:, 0:1], (N, 16))
    w1b = jnp.broadcast_to(routing_weights[:, 1:2], (N, 16))

    xs, wpad = _sc_dispatch(hidden_states, w0b, w1b, pos0, pos1)
    ys = _ffn_call(tile_eid, xs, wpad, W1, W2)
    out = _sc_combine(ys, pos0, pos1)
    return out
